# baseline (speedup 1.0000x reference)
import functools

import jax
import jax.numpy as jnp
from jax import lax
from jax.experimental import pallas as pl
from jax.experimental.pallas import tpu as pltpu
from jax.experimental.pallas import tpu_sc as plsc

B = 64
N = 1024
D = 768
HALF = D // 2
NC = 2
NS = 16
NW = NC * NS
T = B * N
TPW = T // NW
C = 32
STEPS = TPW // C
SPG = 6
NG = (STEPS - 4) // SPG
LPR = HALF // 16


def _sc_pe_add(x2, rows_f, cols_f, pr, pc):
    mesh = plsc.VectorSubcoreMesh(core_axis_name="c", subcore_axis_name="s")

    @functools.partial(
        pl.kernel,
        mesh=mesh,
        out_type=jax.ShapeDtypeStruct((T, D), jnp.float32),
        scratch_types=[
            pltpu.VMEM((TPW,), jnp.int32),
            pltpu.VMEM((TPW,), jnp.int32),
            pltpu.VMEM((C, HALF), jnp.float32),
            pltpu.VMEM((C, HALF), jnp.float32),
            pltpu.VMEM((C, HALF), jnp.float32),
            pltpu.VMEM((C, HALF), jnp.float32),
            pltpu.VMEM((C, D), jnp.float32),
            pltpu.VMEM((C, D), jnp.float32),
            pltpu.VMEM((C, D), jnp.float32),
        ] + [pltpu.SemaphoreType.DMA] * 10,
    )
    def k(x_hbm, rows_hbm, cols_hbm, pr_hbm, pc_hbm, out_hbm,
          idxr_a, idxc_a, prb0, prb1, pcb0, pcb1, xb0, xb1, xb2,
          sgr0, sgr1, sgc0, sgc1, sx0, sx1, sx2, so0, so1, so2):
        wid = lax.axis_index("s") * NC + lax.axis_index("c")
        prb, pcb, xb = (prb0, prb1), (pcb0, pcb1), (xb0, xb1, xb2)
        sgr, sgc = (sgr0, sgr1), (sgc0, sgc1)
        sx, so = (sx0, sx1, sx2), (so0, so1, so2)

        pltpu.sync_copy(rows_hbm.at[pl.ds(wid * TPW, TPW)], idxr_a)
        pltpu.sync_copy(cols_hbm.at[pl.ds(wid * TPW, TPW)], idxc_a)

        def issue_in(s, gp, xp):
            loc = s * C
            pltpu.async_copy(pr_hbm.at[idxr_a.at[pl.ds(loc, C)]], prb[gp],
                             sgr[gp])
            pltpu.async_copy(pc_hbm.at[idxc_a.at[pl.ds(loc, C)]], pcb[gp],
                             sgc[gp])
            pltpu.async_copy(x_hbm.at[pl.ds(wid * TPW + s * C, C)], xb[xp],
                             sx[xp])

        def wait_in(s, gp, xp):
            loc = s * C
            pltpu.make_async_copy(
                pr_hbm.at[idxr_a.at[pl.ds(loc, C)]], prb[gp], sgr[gp]).wait()
            pltpu.make_async_copy(
                pc_hbm.at[idxc_a.at[pl.ds(loc, C)]], pcb[gp], sgc[gp]).wait()
            pltpu.make_async_copy(
                x_hbm.at[pl.ds(wid * TPW + s * C, C)], xb[xp], sx[xp]).wait()

        def issue_out_half(s, xp, h):
            hc = C // 2
            pltpu.async_copy(
                xb[xp].at[pl.ds(h * hc, hc)],
                out_hbm.at[pl.ds(wid * TPW + s * C + h * hc, hc)], so[xp])

        def wait_out(s, xp):
            pltpu.make_async_copy(
                xb[xp], out_hbm.at[pl.ds(wid * TPW + s * C, C)], so[xp]).wait()

        def compute_half(gp, xp, h):
            xbb, prbb, pcbb = xb[xp], prb[gp], pcb[gp]
            hc = C // 2

            def tok(i, carry2):
                for j in range(LPR):
                    plsc.addupdate(xbb.at[i, pl.ds(j * 16, 16)],
                                   prbb[i, pl.ds(j * 16, 16)])
                    plsc.addupdate(xbb.at[i, pl.ds(HALF + j * 16, 16)],
                                   pcbb[i, pl.ds(j * 16, 16)])
                return carry2

            lax.fori_loop(h * hc, (h + 1) * hc, tok, 0)

        def compute_and_out(s, gp, xp):
            def half(h, carry2):
                compute_half(gp, xp, h)
                issue_out_half(s, xp, h)
                return carry2

            lax.fori_loop(0, 2, half, 0)

        def group(g, carry):
            for kk in range(SPG):
                s = g * SPG + kk
                xp, gp = kk % 3, kk % 2
                xp1, gp1 = (kk + 1) % 3, (kk + 1) % 2
                if kk < 2:
                    @pl.when(g > 0)
                    def _():
                        wait_out(s - 2, xp1)
                else:
                    wait_out(s - 2, xp1)
                issue_in(s + 1, gp1, xp1)
                wait_in(s, gp, xp)
                compute_and_out(s, gp, xp)
            return carry

        issue_in(0, 0, 0)
        lax.fori_loop(0, NG, group, 0)
        for s in range(STEPS - 4, STEPS - 1):
            xp, gp = s % 3, s % 2
            xp1, gp1 = (s + 1) % 3, (s + 1) % 2
            wait_out(s - 2, xp1)
            issue_in(s + 1, gp1, xp1)
            wait_in(s, gp, xp)
            compute_and_out(s, gp, xp)
        s = STEPS - 1
        wait_in(s, s % 2, s % 3)
        compute_and_out(s, s % 2, s % 3)
        wait_out(STEPS - 3, (STEPS - 3) % 3)
        wait_out(STEPS - 2, (STEPS - 2) % 3)
        wait_out(STEPS - 1, (STEPS - 1) % 3)

    return k(x2, rows_f, cols_f, pr, pc)


def kernel(x, rows, cols, pe_row, pe_col):
    x2 = x.reshape(T, D)
    rows_f = rows.reshape(T).astype(jnp.int32)
    cols_f = cols.reshape(T).astype(jnp.int32)
    pr = 0.1 * pe_row
    pc = 0.1 * pe_col
    out2 = _sc_pe_add(x2, rows_f, cols_f, pr, pc)
    return out2.reshape(B, N, D)


# speedup vs baseline: 1.0109x; 1.0109x over previous
import functools

import jax
import jax.numpy as jnp
from jax import lax
from jax.experimental import pallas as pl
from jax.experimental.pallas import tpu as pltpu
from jax.experimental.pallas import tpu_sc as plsc

B = 64
N = 1024
D = 768
HALF = D // 2
NC = 2
NS = 16
NW = NC * NS
T = B * N
TPW = T // NW
C = 32
STEPS = TPW // C
SPG = 6
NG = (STEPS - 4) // SPG
LPR = HALF // 16


def _sc_pe_add(x2, rows_f, cols_f, pr, pc):
    mesh = plsc.VectorSubcoreMesh(core_axis_name="c", subcore_axis_name="s")

    @functools.partial(
        pl.kernel,
        mesh=mesh,
        out_type=jax.ShapeDtypeStruct((T, D), jnp.float32),
        scratch_types=[
            pltpu.VMEM((TPW,), jnp.int32),
            pltpu.VMEM((TPW,), jnp.int32),
            pltpu.VMEM((C, HALF), jnp.float32),
            pltpu.VMEM((C, HALF), jnp.float32),
            pltpu.VMEM((C, HALF), jnp.float32),
            pltpu.VMEM((C, HALF), jnp.float32),
            pltpu.VMEM((C, D), jnp.float32),
            pltpu.VMEM((C, D), jnp.float32),
            pltpu.VMEM((C, D), jnp.float32),
        ] + [pltpu.SemaphoreType.DMA] * 10,
    )
    def k(x_hbm, rows_hbm, cols_hbm, pr_hbm, pc_hbm, out_hbm,
          idxr_a, idxc_a, prb0, prb1, pcb0, pcb1, xb0, xb1, xb2,
          sgr0, sgr1, sgc0, sgc1, sx0, sx1, sx2, so0, so1, so2):
        wid = lax.axis_index("s") * NC + lax.axis_index("c")
        prb, pcb, xb = (prb0, prb1), (pcb0, pcb1), (xb0, xb1, xb2)
        sgr, sgc = (sgr0, sgr1), (sgc0, sgc1)
        sx, so = (sx0, sx1, sx2), (so0, so1, so2)

        pltpu.sync_copy(rows_hbm.at[pl.ds(wid * TPW, TPW)], idxr_a)
        pltpu.sync_copy(cols_hbm.at[pl.ds(wid * TPW, TPW)], idxc_a)

        def issue_in(s, gp, xp):
            loc = s * C
            pltpu.async_copy(pr_hbm.at[idxr_a.at[pl.ds(loc, C)]], prb[gp],
                             sgr[gp])
            pltpu.async_copy(pc_hbm.at[idxc_a.at[pl.ds(loc, C)]], pcb[gp],
                             sgc[gp])
            pltpu.async_copy(x_hbm.at[pl.ds(wid * TPW + s * C, C)], xb[xp],
                             sx[xp])

        def wait_in(s, gp, xp):
            loc = s * C
            pltpu.make_async_copy(
                pr_hbm.at[idxr_a.at[pl.ds(loc, C)]], prb[gp], sgr[gp]).wait()
            pltpu.make_async_copy(
                pc_hbm.at[idxc_a.at[pl.ds(loc, C)]], pcb[gp], sgc[gp]).wait()
            pltpu.make_async_copy(
                x_hbm.at[pl.ds(wid * TPW + s * C, C)], xb[xp], sx[xp]).wait()

        def issue_out(s, xp):
            pltpu.async_copy(xb[xp], out_hbm.at[pl.ds(wid * TPW + s * C, C)],
                             so[xp])

        def wait_out(s, xp):
            pltpu.make_async_copy(
                xb[xp], out_hbm.at[pl.ds(wid * TPW + s * C, C)], so[xp]).wait()

        def compute(gp, xp):
            xbb, prbb, pcbb = xb[xp], prb[gp], pcb[gp]

            def tok(i, carry2):
                for j in range(LPR):
                    plsc.addupdate(xbb.at[i, pl.ds(j * 16, 16)],
                                   prbb[i, pl.ds(j * 16, 16)])
                    plsc.addupdate(xbb.at[i, pl.ds(HALF + j * 16, 16)],
                                   pcbb[i, pl.ds(j * 16, 16)])
                return carry2

            lax.fori_loop(0, C, tok, 0)

        def group(g, carry):
            for kk in range(SPG):
                s = g * SPG + kk
                xp, gp = kk % 3, kk % 2
                xp1, gp1 = (kk + 1) % 3, (kk + 1) % 2
                if kk < 2:
                    @pl.when(g > 0)
                    def _():
                        wait_out(s - 2, xp1)
                else:
                    wait_out(s - 2, xp1)
                issue_in(s + 1, gp1, xp1)
                wait_in(s, gp, xp)
                compute(gp, xp)
                issue_out(s, xp)
            return carry

        issue_in(0, 0, 0)
        lax.fori_loop(0, NG, group, 0)
        for s in range(STEPS - 4, STEPS - 1):
            xp, gp = s % 3, s % 2
            xp1, gp1 = (s + 1) % 3, (s + 1) % 2
            wait_out(s - 2, xp1)
            issue_in(s + 1, gp1, xp1)
            wait_in(s, gp, xp)
            compute(gp, xp)
            issue_out(s, xp)
        s = STEPS - 1
        wait_in(s, s % 2, s % 3)
        compute(s % 2, s % 3)
        issue_out(s, s % 3)
        wait_out(STEPS - 3, (STEPS - 3) % 3)
        wait_out(STEPS - 2, (STEPS - 2) % 3)
        wait_out(STEPS - 1, (STEPS - 1) % 3)

    return k(x2, rows_f, cols_f, pr, pc)


def kernel(x, rows, cols, pe_row, pe_col):
    x2 = x.reshape(T, D)
    rows_f = rows.reshape(T).astype(jnp.int32)
    cols_f = cols.reshape(T).astype(jnp.int32)
    pr = 0.1 * pe_row
    pc = 0.1 * pe_col
    out2 = _sc_pe_add(x2, rows_f, cols_f, pr, pc)
    return out2.reshape(B, N, D)
